# baseline (speedup 1.0000x reference)
import functools

import jax
import jax.numpy as jnp
from jax import lax
from jax.experimental import pallas as pl
from jax.experimental.pallas import tpu as pltpu
from jax.experimental.pallas import tpu_sc as plsc

DIM = 64
LANES = 16
IDX_CHUNK = 128


@functools.lru_cache(maxsize=None)
def _make_kernel(batch: int):
    info = plsc.get_sparse_core_info()
    nc, ns = info.num_cores, info.num_subcores
    nw = nc * ns
    bpw = batch // nw
    nch = bpw // IDX_CHUNK
    mesh = plsc.VectorSubcoreMesh(core_axis_name="c", subcore_axis_name="s")

    @functools.partial(
        pl.kernel,
        mesh=mesh,
        out_type=jax.ShapeDtypeStruct((batch,), jnp.float32),
        scratch_types=[
            pltpu.VMEM((nch, IDX_CHUNK), jnp.int32),
            pltpu.VMEM((nch, IDX_CHUNK), jnp.int32),
            pltpu.VMEM((bpw, DIM), jnp.float32),
            pltpu.VMEM((bpw, DIM), jnp.float32),
            pltpu.VMEM((bpw,), jnp.float32),
            pltpu.SemaphoreType.DMA,
        ],
        compiler_params=pltpu.CompilerParams(
            needs_layout_passes=False, use_tc_tiling_on_sc=False),
    )
    def skipgram(center_hbm, context_hbm, inemb_hbm, outemb_hbm, o_hbm,
                 cidx, xidx, arows, crows, ovec, sem):
        wid = lax.axis_index("s") * nc + lax.axis_index("c")
        base = wid * bpw

        for j in range(nch):
            pltpu.sync_copy(center_hbm.at[pl.ds(base + j * IDX_CHUNK, IDX_CHUNK)],
                            cidx.at[j])
            pltpu.sync_copy(context_hbm.at[pl.ds(base + j * IDX_CHUNK, IDX_CHUNK)],
                            xidx.at[j])

        copies = []
        for j in range(nch):
            copies.append(pltpu.async_copy(
                inemb_hbm.at[cidx.at[j]],
                arows.at[pl.ds(j * IDX_CHUNK, IDX_CHUNK)], sem))
            copies.append(pltpu.async_copy(
                outemb_hbm.at[xidx.at[j]],
                crows.at[pl.ds(j * IDX_CHUNK, IDX_CHUNK)], sem))
        for cp in copies:
            cp.wait()

        lane = lax.iota(jnp.int32, LANES)

        def group_body(g, carry):
            row0 = g * LANES
            acc = jnp.zeros((LANES,), jnp.float32)
            for r in range(LANES):
                row = row0 + r
                s = arows[row, pl.ds(0, LANES)] * crows[row, pl.ds(0, LANES)]
                for k in range(1, DIM // LANES):
                    s = s + (arows[row, pl.ds(k * LANES, LANES)]
                             * crows[row, pl.ds(k * LANES, LANES)])
                acc = jnp.where(lane == r, jnp.sum(s), acc)
            ovec[pl.ds(pl.multiple_of(row0, LANES), LANES)] = acc
            return carry

        lax.fori_loop(0, bpw // LANES, group_body, 0)
        pltpu.sync_copy(ovec, o_hbm.at[pl.ds(base, bpw)])

    return skipgram


def kernel(center_words, context_words, in_emb, out_emb):
    (batch,) = center_words.shape
    return _make_kernel(batch)(center_words, context_words, in_emb, out_emb)
